# baseline (speedup 1.0000x reference)
import jax
import jax.numpy as jnp
from jax import lax
from jax.experimental import pallas as pl
from jax.experimental.pallas import tpu as pltpu
from jax.experimental.pallas import tpu_sc as plsc

E = 320000
D = 128
NW = 32
C = 80
CH = E // (NW * C)
NB = 5
LEAD = 2
ROUNDS = CH // NB


def _sc_body(src_hbm, tgt_hbm, eidx_hbm, out_hbm,
             sidx_v, tidx_v, a_v,
             sem_g0, sem_g1, sem_g2, sem_g3, sem_g4,
             sem_o0, sem_o1, sem_o2, sem_o3, sem_o4):
    cid = lax.axis_index("c")
    sid = lax.axis_index("s")
    wid = sid * 2 + cid
    row0 = wid * (CH * C)

    sem_g = [sem_g0, sem_g1, sem_g2, sem_g3, sem_g4]
    sem_o = [sem_o0, sem_o1, sem_o2, sem_o3, sem_o4]

    pltpu.sync_copy(eidx_hbm.at[pl.ds(wid * (CH * C), CH * C)], sidx_v)
    pltpu.sync_copy(eidx_hbm.at[pl.ds(E + wid * (CH * C), CH * C)], tidx_v)

    def fire(chunk, b):
        sl = pl.ds(chunk * C, C)
        pltpu.async_copy(src_hbm.at[sidx_v.at[sl]],
                         a_v.at[b, pl.ds(0, C)], sem_g[b])
        pltpu.async_copy(tgt_hbm.at[tidx_v.at[sl]],
                         a_v.at[b, pl.ds(C, C)], sem_g[b])

    def wait_gathers(chunk, b):
        sl = pl.ds(chunk * C, C)
        pltpu.make_async_copy(src_hbm.at[sidx_v.at[sl]],
                              a_v.at[b, pl.ds(0, C)], sem_g[b]).wait()
        pltpu.make_async_copy(tgt_hbm.at[tidx_v.at[sl]],
                              a_v.at[b, pl.ds(C, C)], sem_g[b]).wait()

    def wait_out(chunk, b):
        pltpu.make_async_copy(a_v.at[b, pl.ds(0, C)],
                              out_hbm.at[pl.ds(row0 + chunk * C, C)],
                              sem_o[b]).wait()

    for b in range(LEAD):
        fire(b, b)

    def round_body(r, carry):
        base = r * NB
        for b in range(NB):
            chunk = base + b

            wait_gathers(chunk, b)

            def row_body(row, c2):
                for j in range(D // 16):
                    sl = pl.ds(j * 16, 16)
                    a_v[b, row, sl] = a_v[b, row, sl] * a_v[b, C + row, sl]
                return c2

            lax.fori_loop(0, C, row_body, 0, unroll=2)

            pltpu.async_copy(a_v.at[b, pl.ds(0, C)],
                             out_hbm.at[pl.ds(row0 + chunk * C, C)], sem_o[b])

            nb_ = (b + LEAD) % NB

            @pl.when(chunk < CH - LEAD)
            def _():
                @pl.when(chunk >= NB - LEAD)
                def _():
                    wait_out(chunk + LEAD - NB, nb_)
                fire(chunk + LEAD, nb_)
        return carry

    lax.fori_loop(0, ROUNDS, round_body, 0)

    for k in range(NB):
        chunk = CH - NB + k
        wait_out(chunk, chunk % NB)


def kernel(node_src_feats, node_tgt_feats, edge_ids):
    eids = edge_ids.astype(jnp.int32).reshape(2 * E)

    mesh = plsc.VectorSubcoreMesh(core_axis_name="c", subcore_axis_name="s")
    f = pl.kernel(
        _sc_body,
        mesh=mesh,
        out_type=jax.ShapeDtypeStruct((E, D), jnp.float32),
        scratch_types=[
            pltpu.VMEM((CH * C,), jnp.int32),
            pltpu.VMEM((CH * C,), jnp.int32),
            pltpu.VMEM((NB, 2 * C, D), jnp.float32),
        ] + [pltpu.SemaphoreType.DMA] * (2 * NB),
    )
    return f(node_src_feats, node_tgt_feats, eids)


# speedup vs baseline: 1.0587x; 1.0587x over previous
import jax
import jax.numpy as jnp
from jax import lax
from jax.experimental import pallas as pl
from jax.experimental.pallas import tpu as pltpu
from jax.experimental.pallas import tpu_sc as plsc

E = 320000
D = 128
NW = 32
C = 80
CH = E // (NW * C)
NB = 5
LEAD = 3
ROUNDS = CH // NB


def _sc_body(src_hbm, tgt_hbm, eidx_hbm, out_hbm,
             sidx_v, tidx_v, a_v,
             sem_g0, sem_g1, sem_g2, sem_g3, sem_g4,
             sem_o0, sem_o1, sem_o2, sem_o3, sem_o4):
    cid = lax.axis_index("c")
    sid = lax.axis_index("s")
    wid = sid * 2 + cid
    row0 = wid * (CH * C)

    sem_g = [sem_g0, sem_g1, sem_g2, sem_g3, sem_g4]
    sem_o = [sem_o0, sem_o1, sem_o2, sem_o3, sem_o4]

    pltpu.sync_copy(eidx_hbm.at[pl.ds(wid * (CH * C), CH * C)], sidx_v)
    pltpu.sync_copy(eidx_hbm.at[pl.ds(E + wid * (CH * C), CH * C)], tidx_v)

    def fire(chunk, b):
        sl = pl.ds(chunk * C, C)
        pltpu.async_copy(src_hbm.at[sidx_v.at[sl]],
                         a_v.at[b, pl.ds(0, C)], sem_g[b])
        pltpu.async_copy(tgt_hbm.at[tidx_v.at[sl]],
                         a_v.at[b, pl.ds(C, C)], sem_g[b])

    def wait_gathers(chunk, b):
        sl = pl.ds(chunk * C, C)
        pltpu.make_async_copy(src_hbm.at[sidx_v.at[sl]],
                              a_v.at[b, pl.ds(0, C)], sem_g[b]).wait()
        pltpu.make_async_copy(tgt_hbm.at[tidx_v.at[sl]],
                              a_v.at[b, pl.ds(C, C)], sem_g[b]).wait()

    def wait_out(chunk, b):
        pltpu.make_async_copy(a_v.at[b, pl.ds(0, C)],
                              out_hbm.at[pl.ds(row0 + chunk * C, C)],
                              sem_o[b]).wait()

    for b in range(LEAD):
        fire(b, b)

    def round_body(r, carry):
        base = r * NB
        for b in range(NB):
            chunk = base + b

            wait_gathers(chunk, b)

            def row_body(row, c2):
                for j in range(D // 16):
                    sl = pl.ds(j * 16, 16)
                    a_v[b, row, sl] = a_v[b, row, sl] * a_v[b, C + row, sl]
                return c2

            lax.fori_loop(0, C, row_body, 0, unroll=2)

            pltpu.async_copy(a_v.at[b, pl.ds(0, C)],
                             out_hbm.at[pl.ds(row0 + chunk * C, C)], sem_o[b])

            nb_ = (b + LEAD) % NB

            @pl.when(chunk < CH - LEAD)
            def _():
                @pl.when(chunk >= NB - LEAD)
                def _():
                    wait_out(chunk + LEAD - NB, nb_)
                fire(chunk + LEAD, nb_)
        return carry

    lax.fori_loop(0, ROUNDS, round_body, 0)

    for k in range(NB):
        chunk = CH - NB + k
        wait_out(chunk, chunk % NB)


def kernel(node_src_feats, node_tgt_feats, edge_ids):
    eids = edge_ids.astype(jnp.int32).reshape(2 * E)

    mesh = plsc.VectorSubcoreMesh(core_axis_name="c", subcore_axis_name="s")
    f = pl.kernel(
        _sc_body,
        mesh=mesh,
        out_type=jax.ShapeDtypeStruct((E, D), jnp.float32),
        scratch_types=[
            pltpu.VMEM((CH * C,), jnp.int32),
            pltpu.VMEM((CH * C,), jnp.int32),
            pltpu.VMEM((NB, 2 * C, D), jnp.float32),
        ] + [pltpu.SemaphoreType.DMA] * (2 * NB),
    )
    return f(node_src_feats, node_tgt_feats, eids)
